# baseline (speedup 1.0000x reference)
import math

import jax
import jax.numpy as jnp
from jax import lax
from jax.experimental import pallas as pl
from jax.experimental.pallas import tpu as pltpu
from jax.experimental.pallas import tpu_sc as plsc

N_NODES = 10000
N_EDGES = 320000
D = 128
D_ATTR = 16
N_BASIS = 8
HID = 8

NC = 2
NS = 16
LANES = 16
DH = D // NC
K = 128
KP = K // 2
CHUNKS = 160
GCH = 16
GROUPS = CHUNKS // GCH
EPW = K * CHUNKS
E_PAD = NS * EPW
E_HALF = E_PAD // 2
N_ACC = 10240
NPW = N_ACC // NS

_INV_SQRT_D = 1.0 / math.sqrt(float(D))
_INV_SQRT_HID = 1.0 / math.sqrt(float(HID))
_INV_SQRT_NB = 1.0 / math.sqrt(float(N_BASIS))
_INV_SQRT_SC = 1.0 / math.sqrt(float(D * D_ATTR))


def _node_body(h_ref, x_ref, wl1_ref, wsc_ref, h1_ref, sc_ref):
    hb = h_ref[...]
    xb = x_ref[...]
    h1_ref[...] = jnp.dot(hb, wl1_ref[...],
                          preferred_element_type=jnp.float32) * _INV_SQRT_D
    p = jnp.concatenate([hb * xb[:, v:v + 1] for v in range(D_ATTR)], axis=1)
    sc_ref[...] = jnp.dot(p, wsc_ref[...],
                          preferred_element_type=jnp.float32) * _INV_SQRT_SC


def _edge_weight(elem, sh, r, wfc0, wfc1, wr0, br0, wr1, br1, wr2, br2,
                 wr3, br3):
    z = jnp.dot(elem, wfc0, preferred_element_type=jnp.float32)
    z = jax.nn.softplus(z * _INV_SQRT_NB) - math.log(2.0)
    w = jnp.dot(z, wfc1, preferred_element_type=jnp.float32) * _INV_SQRT_HID
    n = (lax.broadcasted_iota(jnp.int32, (1, N_BASIS), 1).astype(jnp.float32)
         + 1.0) * math.pi
    basis = jnp.sin(n * r) / r
    a = jax.nn.silu(jnp.dot(basis, wr0,
                            preferred_element_type=jnp.float32) + br0)
    a = jax.nn.silu(jnp.dot(a, wr1, preferred_element_type=jnp.float32) + br1)
    a = jax.nn.silu(jnp.dot(a, wr2, preferred_element_type=jnp.float32) + br2)
    att = jnp.dot(a, wr3, preferred_element_type=jnp.float32) + br3
    return w * (sh * att)


def _edge_body(elem_lo, sh_lo, r_lo, elem_hi, sh_hi, r_hi,
               wfc0_ref, wfc1_ref, wr0_ref, br0_ref, wr1_ref, br1_ref,
               wr2_ref, br2_ref, wr3_ref, br3_ref, wfp_ref):
    ws = (wfc0_ref[...], wfc1_ref[...], wr0_ref[...], br0_ref[...],
          wr1_ref[...], br1_ref[...], wr2_ref[...], br2_ref[...],
          wr3_ref[...], br3_ref[...])
    wf_lo = _edge_weight(elem_lo[...], sh_lo[...], r_lo[...], *ws)
    wf_hi = _edge_weight(elem_hi[...], sh_hi[...], r_hi[...], *ws)
    wfp_ref[0] = jnp.concatenate([wf_lo[:, :DH], wf_hi[:, :DH]], axis=1)
    wfp_ref[1] = jnp.concatenate([wf_lo[:, DH:], wf_hi[:, DH:]], axis=1)


def _sc_body(h1_hbm, wf_hbm, src_hbm, dst_hbm, out_hbm,
             sb, db, gi, ssc0, ssc1, dsc, wfi0, wfi1,
             rows0, rows1, wv0, wv1, rh, acc,
             sem_i, sem_g0, sem_g1, sem_w0, sem_w1):
    c = lax.axis_index("c")
    s = lax.axis_index("s")

    zv = jnp.zeros((LANES,), jnp.float32)

    @pl.loop(0, K)
    def _(r):
        for k in range(DH // LANES):
            rh[(r, pl.ds(k * LANES, LANES))] = zv

    for t in range(NPW // K):
        pltpu.sync_copy(rh, acc.at[pl.ds(s * NPW + t * K, K)])
    plsc.subcore_barrier()

    rbase = s * (EPW // 2)
    wfbase = c * E_HALF + rbase
    iota = lax.iota(jnp.int32, LANES)

    def fill_issue(g, j, ssc, wfi, rows, wv, sem_g, sem_w):
        for k in range(K // LANES):
            ssc[pl.ds(k * LANES, LANES)] = sb[(j, pl.ds(k * LANES, LANES))]
        q = g * GCH + j
        for k in range(KP // LANES):
            wfi[pl.ds(k * LANES, LANES)] = iota + (
                wfbase + q * KP + k * LANES)
        pltpu.async_copy(h1_hbm.at[ssc], rows, sem_g)
        pltpu.async_copy(wf_hbm.at[wfi], wv, sem_w)

    def consume(j, ssc, wfi, rows, wv, sem_g, sem_w):
        pltpu.make_async_copy(h1_hbm.at[ssc], rows, sem_g).wait()
        pltpu.make_async_copy(wf_hbm.at[wfi], wv, sem_w).wait()
        coff = c * DH

        @pl.loop(0, KP)
        def _(m):
            for k in range(DH // LANES):
                rh[(2 * m, pl.ds(k * LANES, LANES))] = (
                    rows[(2 * m, pl.ds(coff + k * LANES, LANES))]
                    * wv[(m, pl.ds(k * LANES, LANES))])
                rh[(2 * m + 1, pl.ds(k * LANES, LANES))] = (
                    rows[(2 * m + 1, pl.ds(coff + k * LANES, LANES))]
                    * wv[(m, pl.ds(DH + k * LANES, LANES))])

        for k in range(K // LANES):
            dsc[pl.ds(k * LANES, LANES)] = db[(j, pl.ds(k * LANES, LANES))]
        pltpu.sync_copy(rh, acc.at[dsc], add=True)

    @pl.loop(0, GROUPS)
    def _(g):
        gi[pl.ds(0, LANES)] = iota + (s * CHUNKS + g * GCH)
        pltpu.async_copy(src_hbm.at[gi], sb, sem_i)
        pltpu.async_copy(dst_hbm.at[gi], db, sem_i)
        pltpu.make_async_copy(src_hbm.at[gi], sb, sem_i).wait()
        pltpu.make_async_copy(dst_hbm.at[gi], db, sem_i).wait()

        @pl.loop(0, GCH)
        def _(j):
            fill_issue(g, j, ssc0, wfi0, rows0, wv0, sem_g0, sem_w0)
            consume(j, ssc0, wfi0, rows0, wv0, sem_g0, sem_w0)

    plsc.subcore_barrier()
    for t in range(NPW // K):
        pltpu.sync_copy(acc.at[pl.ds(s * NPW + t * K, K)], rh)
        pltpu.sync_copy(rh, out_hbm.at[c, pl.ds(s * NPW + t * K, K)])


def _final_body(parts_ref, sc_ref, wl2_ref, out_ref):
    hagg = jnp.concatenate([parts_ref[0], parts_ref[1]], axis=1)
    out_ref[...] = jnp.dot(hagg, wl2_ref[...],
                           preferred_element_type=jnp.float32) * _INV_SQRT_D \
        + sc_ref[...]


def kernel(x, h, edge_length_embeddings, edge_sh, edge_index, r_ijs,
           W_lin1, Wfc0, Wfc1, Wr0, br0, Wr1, br1, Wr2, br2, Wr3, br3,
           W_lin2, W_sc):
    idx = edge_index.astype(jnp.int32)
    pad = E_PAD - N_EDGES

    def pair_idx(a):
        a = jnp.pad(a, (0, pad))
        lo = a[:E_HALF].reshape(NS, CHUNKS, KP)
        hi = a[E_HALF:].reshape(NS, CHUNKS, KP)
        return jnp.stack([lo, hi], axis=-1).reshape(NS * CHUNKS, K)

    src2d = pair_idx(idx[1])
    dst2d = pair_idx(idx[0])
    elem_p = jnp.pad(edge_length_embeddings, ((0, pad), (0, 0)))
    sh_p = jnp.pad(edge_sh, ((0, pad), (0, 0)))
    r_p = jnp.pad(r_ijs.reshape(N_EDGES, 1), ((0, pad), (0, 0)),
                  constant_values=1.0)
    wsc2d = W_sc.transpose(1, 0, 2).reshape(D_ATTR * D, D)
    br0_ = br0.reshape(1, HID)
    br1_ = br1.reshape(1, HID)
    br2_ = br2.reshape(1, HID)
    br3_ = br3.reshape(1, 1)

    nblk = 1000
    h1, sc = pl.pallas_call(
        _node_body,
        grid=(N_NODES // nblk,),
        in_specs=[
            pl.BlockSpec((nblk, D), lambda i: (i, 0)),
            pl.BlockSpec((nblk, D_ATTR), lambda i: (i, 0)),
            pl.BlockSpec((D, D), lambda i: (0, 0)),
            pl.BlockSpec((D_ATTR * D, D), lambda i: (0, 0)),
        ],
        out_specs=[
            pl.BlockSpec((nblk, D), lambda i: (i, 0)),
            pl.BlockSpec((nblk, D), lambda i: (i, 0)),
        ],
        out_shape=[
            jax.ShapeDtypeStruct((N_NODES, D), jnp.float32),
            jax.ShapeDtypeStruct((N_NODES, D), jnp.float32),
        ],
    )(h, x, W_lin1, wsc2d)

    eblk = 2048
    hi_off = E_HALF // eblk
    small = lambda a, b: pl.BlockSpec((a, b), lambda i: (0, 0))
    wfp = pl.pallas_call(
        _edge_body,
        grid=(E_HALF // eblk,),
        in_specs=[
            pl.BlockSpec((eblk, N_BASIS), lambda i: (i, 0)),
            pl.BlockSpec((eblk, 1), lambda i: (i, 0)),
            pl.BlockSpec((eblk, 1), lambda i: (i, 0)),
            pl.BlockSpec((eblk, N_BASIS), lambda i: (i + hi_off, 0)),
            pl.BlockSpec((eblk, 1), lambda i: (i + hi_off, 0)),
            pl.BlockSpec((eblk, 1), lambda i: (i + hi_off, 0)),
            small(N_BASIS, HID), small(HID, D),
            small(N_BASIS, HID), small(1, HID),
            small(HID, HID), small(1, HID),
            small(HID, HID), small(1, HID),
            small(HID, 1), small(1, 1),
        ],
        out_specs=pl.BlockSpec((NC, eblk, D), lambda i: (0, i, 0)),
        out_shape=jax.ShapeDtypeStruct((NC, E_HALF, D), jnp.float32),
    )(elem_p, sh_p, r_p, elem_p, sh_p, r_p, Wfc0, Wfc1, Wr0, br0_,
      Wr1, br1_, Wr2, br2_, Wr3, br3_)

    mesh = plsc.VectorSubcoreMesh(core_axis_name="c", subcore_axis_name="s")
    sc_call = pl.kernel(
        _sc_body,
        out_type=jax.ShapeDtypeStruct((NC, N_ACC, DH), jnp.float32),
        mesh=mesh,
        scratch_types=[
            pltpu.VMEM((GCH, K), jnp.int32),
            pltpu.VMEM((GCH, K), jnp.int32),
            pltpu.VMEM((LANES,), jnp.int32),
            pltpu.VMEM((K,), jnp.int32),
            pltpu.VMEM((K,), jnp.int32),
            pltpu.VMEM((K,), jnp.int32),
            pltpu.VMEM((KP,), jnp.int32),
            pltpu.VMEM((KP,), jnp.int32),
            pltpu.VMEM((K, D), jnp.float32),
            pltpu.VMEM((K, D), jnp.float32),
            pltpu.VMEM((KP, D), jnp.float32),
            pltpu.VMEM((KP, D), jnp.float32),
            pltpu.VMEM((K, DH), jnp.float32),
            pltpu.VMEM_SHARED((N_ACC, DH), jnp.float32),
            pltpu.SemaphoreType.DMA,
            pltpu.SemaphoreType.DMA,
            pltpu.SemaphoreType.DMA,
            pltpu.SemaphoreType.DMA,
            pltpu.SemaphoreType.DMA,
        ],
    )
    parts = sc_call(h1, wfp.reshape(NC * E_HALF, D), src2d, dst2d)

    out = pl.pallas_call(
        _final_body,
        grid=(N_NODES // nblk,),
        in_specs=[
            pl.BlockSpec((NC, nblk, DH), lambda i: (0, i, 0)),
            pl.BlockSpec((nblk, D), lambda i: (i, 0)),
            pl.BlockSpec((D, D), lambda i: (0, 0)),
        ],
        out_specs=pl.BlockSpec((nblk, D), lambda i: (i, 0)),
        out_shape=jax.ShapeDtypeStruct((N_NODES, D), jnp.float32),
    )(parts, sc, W_lin2)
    return out


# speedup vs baseline: 1.2953x; 1.2953x over previous
import math

import jax
import jax.numpy as jnp
from jax import lax
from jax.experimental import pallas as pl
from jax.experimental.pallas import tpu as pltpu
from jax.experimental.pallas import tpu_sc as plsc

N_NODES = 10000
N_EDGES = 320000
D = 128
D_ATTR = 16
N_BASIS = 8
HID = 8

NC = 2
NS = 16
LANES = 16
DH = D // NC
K = 128
KP = K // 2
CHUNKS = 160
GCH = 16
GROUPS = CHUNKS // GCH
EPW = K * CHUNKS
E_PAD = NS * EPW
E_HALF = E_PAD // 2
N_ACC = 10240
NPW = N_ACC // NS

_INV_SQRT_D = 1.0 / math.sqrt(float(D))
_INV_SQRT_HID = 1.0 / math.sqrt(float(HID))
_INV_SQRT_NB = 1.0 / math.sqrt(float(N_BASIS))
_INV_SQRT_SC = 1.0 / math.sqrt(float(D * D_ATTR))


def _node_body(h_ref, x_ref, wl1_ref, wsc_ref, h1_ref, sc_ref):
    hb = h_ref[...]
    xb = x_ref[...]
    h1_ref[...] = jnp.dot(hb, wl1_ref[...],
                          preferred_element_type=jnp.float32) * _INV_SQRT_D
    p = jnp.concatenate([hb * xb[:, v:v + 1] for v in range(D_ATTR)], axis=1)
    sc_ref[...] = jnp.dot(p, wsc_ref[...],
                          preferred_element_type=jnp.float32) * _INV_SQRT_SC


def _edge_weight(elem, sh, r, wfc0, wfc1, wr0, br0, wr1, br1, wr2, br2,
                 wr3, br3):
    z = jnp.dot(elem, wfc0, preferred_element_type=jnp.float32)
    z = jax.nn.softplus(z * _INV_SQRT_NB) - math.log(2.0)
    w = jnp.dot(z, wfc1, preferred_element_type=jnp.float32) * _INV_SQRT_HID
    n = (lax.broadcasted_iota(jnp.int32, (1, N_BASIS), 1).astype(jnp.float32)
         + 1.0) * math.pi
    basis = jnp.sin(n * r) / r
    a = jax.nn.silu(jnp.dot(basis, wr0,
                            preferred_element_type=jnp.float32) + br0)
    a = jax.nn.silu(jnp.dot(a, wr1, preferred_element_type=jnp.float32) + br1)
    a = jax.nn.silu(jnp.dot(a, wr2, preferred_element_type=jnp.float32) + br2)
    att = jnp.dot(a, wr3, preferred_element_type=jnp.float32) + br3
    return w * (sh * att)


def _edge_body(elem_lo, sh_lo, r_lo, elem_hi, sh_hi, r_hi,
               wfc0_ref, wfc1_ref, wr0_ref, br0_ref, wr1_ref, br1_ref,
               wr2_ref, br2_ref, wr3_ref, br3_ref, wfp_ref):
    ws = (wfc0_ref[...], wfc1_ref[...], wr0_ref[...], br0_ref[...],
          wr1_ref[...], br1_ref[...], wr2_ref[...], br2_ref[...],
          wr3_ref[...], br3_ref[...])
    wf_lo = _edge_weight(elem_lo[...], sh_lo[...], r_lo[...], *ws)
    wf_hi = _edge_weight(elem_hi[...], sh_hi[...], r_hi[...], *ws)
    wfp_ref[0] = jnp.concatenate([wf_lo[:, :DH], wf_hi[:, :DH]], axis=1)
    wfp_ref[1] = jnp.concatenate([wf_lo[:, DH:], wf_hi[:, DH:]], axis=1)


def _final_body(hagg_ref, sc_ref, wl2_ref, out_ref):
    out_ref[...] = jnp.dot(hagg_ref[...], wl2_ref[...],
                           preferred_element_type=jnp.float32) * _INV_SQRT_D \
        + sc_ref[...]


def kernel(x, h, edge_length_embeddings, edge_sh, edge_index, r_ijs,
           W_lin1, Wfc0, Wfc1, Wr0, br0, Wr1, br1, Wr2, br2, Wr3, br3,
           W_lin2, W_sc):
    wsc2d = W_sc.transpose(1, 0, 2).reshape(D_ATTR * D, D)
    nblk = 1000
    h1, sc = pl.pallas_call(
        _node_body,
        grid=(N_NODES // nblk,),
        in_specs=[
            pl.BlockSpec((nblk, D), lambda i: (i, 0)),
            pl.BlockSpec((nblk, D_ATTR), lambda i: (i, 0)),
            pl.BlockSpec((D, D), lambda i: (0, 0)),
            pl.BlockSpec((D_ATTR * D, D), lambda i: (0, 0)),
        ],
        out_specs=[
            pl.BlockSpec((nblk, D), lambda i: (i, 0)),
            pl.BlockSpec((nblk, D), lambda i: (i, 0)),
        ],
        out_shape=[
            jax.ShapeDtypeStruct((N_NODES, D), jnp.float32),
            jax.ShapeDtypeStruct((N_NODES, D), jnp.float32),
        ],
    )(h, x, W_lin1, wsc2d)

    z = jax.nn.softplus(
        edge_length_embeddings @ Wfc0 * _INV_SQRT_NB) - math.log(2.0)
    weight = z @ Wfc1 * _INV_SQRT_HID
    n = jnp.arange(1, N_BASIS + 1, dtype=jnp.float32) * math.pi
    r_ = r_ijs[:, None]
    basis = jnp.sin(n * r_) / r_
    a = jax.nn.silu(basis @ Wr0 + br0)
    a = jax.nn.silu(a @ Wr1 + br1)
    a = jax.nn.silu(a @ Wr2 + br2)
    att = a @ Wr3 + br3
    h_src = jnp.take(h1, edge_index[1], axis=0)
    h_agg = jax.ops.segment_sum(h_src * edge_sh * weight * att,
                                edge_index[0], num_segments=N_NODES)

    out = pl.pallas_call(
        _final_body,
        grid=(N_NODES // nblk,),
        in_specs=[
            pl.BlockSpec((nblk, D), lambda i: (i, 0)),
            pl.BlockSpec((nblk, D), lambda i: (i, 0)),
            pl.BlockSpec((D, D), lambda i: (0, 0)),
        ],
        out_specs=pl.BlockSpec((nblk, D), lambda i: (i, 0)),
        out_shape=jax.ShapeDtypeStruct((N_NODES, D), jnp.float32),
    )(h_agg, sc, W_lin2)
    return out
